# baseline (speedup 1.0000x reference)
import functools

import jax
import jax.numpy as jnp
from jax import lax
from jax.experimental import pallas as pl
from jax.experimental.pallas import tpu as pltpu
from jax.experimental.pallas import tpu_sc as plsc

N = 10000
E = 320000
D = 128
DE = 16
G = 64
STEPS = 3
BLK = 1000
EBLK = 4096
RB = 128

NW = 32
E_P = 344064
ESZ = E_P // NW
EB1 = 384
NB1 = ESZ // EB1
CG = 16
ESH = 2
W = 8
EPS2 = E_P // ESH
EB2 = 2048
NB2 = EPS2 // EB2
SROWS = E_P // 16
ACCW = (N + 2) * W

_CP = pltpu.CompilerParams(needs_layout_passes=False)


def _lrelu(v):
    return jnp.where(v > 0, v, 0.01 * v)


def _p1_body(y_hbm, e_hbm, src_hbm, out_hbm, srcf, idxg, ybuf0, ybuf1, sem, osem):
    cid = lax.axis_index("c")
    sid = lax.axis_index("s")
    wid = sid * 2 + cid

    def half(b, buf, first):
        e0 = wid * ESZ + b * EB1
        pltpu.sync_copy(src_hbm.at[pl.ds(e0, EB1)], srcf)
        for r in range(EB1 // 128):
            for i in range(8):
                idxg[r, pl.ds(i * 16, 16)] = srcf[pl.ds(r * 128 + i * 16, 16)]
        @pl.when(jnp.logical_not(first))
        def _():
            pltpu.make_async_copy(e_hbm.at[pl.ds(e0, EB1)], buf, osem).wait()
        pltpu.sync_copy(e_hbm.at[pl.ds(e0, EB1)], buf)
        copies = [
            pltpu.async_copy(y_hbm.at[idxg.at[r]],
                             buf.at[pl.ds(r * 128, 128)], sem, add=True)
            for r in range(EB1 // 128)
        ]
        for cp in copies:
            cp.wait()
        pltpu.async_copy(buf, out_hbm.at[pl.ds(e0, EB1)], osem)

    def blk(b2, carry):
        half(b2 * 2, ybuf0, b2 == 0)
        half(b2 * 2 + 1, ybuf1, b2 == 0)
        return carry
    lax.fori_loop(0, NB1 // 2, blk, 0)
    pltpu.make_async_copy(e_hbm.at[pl.ds(0, EB1)], ybuf0, osem).wait()
    pltpu.make_async_copy(e_hbm.at[pl.ds(0, EB1)], ybuf1, osem).wait()


def _p1(y, e_p, src_p):
    kfn = functools.partial(
        pl.kernel,
        mesh=plsc.VectorSubcoreMesh(core_axis_name="c", subcore_axis_name="s"),
        compiler_params=_CP,
        out_type=jax.ShapeDtypeStruct((E_P, D), jnp.float32),
        scratch_types=[
            pltpu.VMEM((EB1,), jnp.int32),
            pltpu.VMEM((EB1 // 128, 128), jnp.int32),
            pltpu.VMEM((EB1, D), jnp.float32),
            pltpu.VMEM((EB1, D), jnp.float32),
            pltpu.SemaphoreType.DMA,
            pltpu.SemaphoreType.DMA,
        ],
    )(_p1_body)
    return kfn(y, e_p, src_p)


def _p2_body(s_hbm, dst_hbm, out_hbm, acc, dstf, sbuf):
    cid = lax.axis_index("c")
    sid = lax.axis_index("s")
    wid = sid * 2 + cid
    cg = wid % CG
    es = wid // CG
    i16 = lax.iota(jnp.int32, 16)
    neg = jnp.full((16,), -jnp.inf, jnp.float32)

    def init_body(i, carry):
        acc[pl.ds(i * 16, 16)] = neg
        return carry
    lax.fori_loop(0, ACCW // 16, init_body, 0)

    def blk(b, carry):
        e0 = es * EPS2 + b * EB2
        pltpu.sync_copy(dst_hbm.at[pl.ds(e0, EB2)], dstf)
        row0 = pl.multiple_of(e0 // 16, 128)
        pltpu.sync_copy(s_hbm.at[cg, pl.ds(row0, EB2 // 16)], sbuf)

        def pair(j, eidx0):
            i16_ = lax.iota(jnp.int32, 16)
            col_ = i16_ & 7
            swd_ = 1 - 2 * (i16_ // 8)
            for u in range(8):
                eidx = eidx0 + 2 * u
                eidx_sw = eidx + swd_
                fl = eidx * 8 + col_
                rows = plsc.load_gather(dstf, [eidx])
                r_sw = plsc.load_gather(dstf, [eidx_sw])
                s = plsc.load_gather(sbuf, [fl >> 7, fl & 127])
                s_sw = plsc.load_gather(sbuf, [fl >> 7, (fl & 127) + swd_ * 8])
                dup = rows == r_sw
                s = jnp.where(dup, jnp.maximum(s, s_sw), s)
                fidx = rows * 8 + col_
                cur = plsc.load_gather(acc, [fidx])
                s = jnp.maximum(s, cur)
                plsc.store_scatter(acc, [fidx], s)
            return eidx0 + 16
        lax.fori_loop(0, EB2 // 16, pair, i16 // 8)
        return carry
    lax.fori_loop(0, NB2, blk, 0)
    pltpu.sync_copy(acc.at[pl.ds(0, N * W)], out_hbm.at[pl.ds(wid * N * W, N * W)])


def _p2(s_p, dst_p):
    kfn = functools.partial(
        pl.kernel,
        mesh=plsc.VectorSubcoreMesh(core_axis_name="c", subcore_axis_name="s"),
        compiler_params=_CP,
        out_type=jax.ShapeDtypeStruct((NW * N * W,), jnp.float32),
        scratch_types=[
            pltpu.VMEM((ACCW,), jnp.float32),
            pltpu.VMEM((EB2,), jnp.int32),
            pltpu.VMEM((EB2 // 16, 128), jnp.float32),
        ],
    )(_p2_body)
    return kfn(s_p, dst_p)


def _ymat_body(x_ref, w_ref, b_ref, out_ref):
    out_ref[...] = (jnp.dot(x_ref[...], w_ref[...],
                            preferred_element_type=jnp.float32) + b_ref[...])


def _ymat(x, w, b):
    return pl.pallas_call(
        _ymat_body,
        grid=(N // BLK,),
        in_specs=[
            pl.BlockSpec((BLK, D), lambda i: (i, 0)),
            pl.BlockSpec((D, D), lambda i: (0, 0)),
            pl.BlockSpec((1, D), lambda i: (0, 0)),
        ],
        out_specs=pl.BlockSpec((BLK, D), lambda i: (i, 0)),
        out_shape=jax.ShapeDtypeStruct((N, D), jnp.float32),
    )(x, w, b)


def _emat_body(a_ref, w_ref, out_ref):
    out_ref[...] = jnp.dot(a_ref[...], w_ref[...],
                           preferred_element_type=jnp.float32)


def _emat(attr, w):
    return pl.pallas_call(
        _emat_body,
        grid=(E_P // EBLK,),
        in_specs=[
            pl.BlockSpec((EBLK, DE), lambda i: (i, 0)),
            pl.BlockSpec((DE, D), lambda i: (0, 0)),
        ],
        out_specs=pl.BlockSpec((EBLK, D), lambda i: (i, 0)),
        out_shape=jax.ShapeDtypeStruct((E_P, D), jnp.float32),
    )(attr, w)


def _repack_body(s_ref, out_ref):
    s = s_ref[...]
    s4 = s.reshape(RB, 16, CG, W)
    out_ref[...] = jnp.transpose(s4, (2, 0, 1, 3)).reshape(CG, RB, 128)


def _repack(s):
    return pl.pallas_call(
        _repack_body,
        grid=(SROWS // RB,),
        in_specs=[pl.BlockSpec((RB * 16, D), lambda i: (i, 0))],
        out_specs=pl.BlockSpec((CG, RB, 128), lambda i: (0, i, 0)),
        out_shape=jax.ShapeDtypeStruct((CG, SROWS, 128), jnp.float32),
    )(s)


def _update_x_body(x_ref, m_ref, xgn_ref, wa0_ref, wa1_ref, wa2_ref, ba_ref, out_ref):
    x = x_ref[...]
    m = jnp.max(m_ref[...], axis=0)
    agg = jnp.where(m > -jnp.inf, _lrelu(m), 0.0)
    acc = jnp.dot(x, wa0_ref[...], preferred_element_type=jnp.float32)
    acc += jnp.dot(xgn_ref[...], wa1_ref[...], preferred_element_type=jnp.float32)
    acc += jnp.dot(agg, wa2_ref[...], preferred_element_type=jnp.float32)
    acc += ba_ref[...]
    out_ref[...] = _lrelu(acc) + x


def _update_x(x, m2, xgn, wa0, wa1, wa2, ba):
    return pl.pallas_call(
        _update_x_body,
        grid=(N // BLK,),
        in_specs=[
            pl.BlockSpec((BLK, D), lambda i: (i, 0)),
            pl.BlockSpec((ESH, BLK, D), lambda i: (0, i, 0)),
            pl.BlockSpec((BLK, D), lambda i: (i, 0)),
            pl.BlockSpec((D, D), lambda i: (0, 0)),
            pl.BlockSpec((D, D), lambda i: (0, 0)),
            pl.BlockSpec((D, D), lambda i: (0, 0)),
            pl.BlockSpec((1, D), lambda i: (0, 0)),
        ],
        out_specs=pl.BlockSpec((BLK, D), lambda i: (i, 0)),
        out_shape=jax.ShapeDtypeStruct((N, D), jnp.float32),
    )(x, m2, xgn, wa0, wa1, wa2, ba)


def kernel(x, xg_init, edge_attr, Wm, bm, Wa, ba, Wgate, bgate, Wfeat, bfeat, Wt, bt,
           edge_index, batch_ind, num_graphs, data_lens):
    src_p = jnp.pad(edge_index[0], (0, E_P - E))
    dst_p = jnp.pad(edge_index[1], (0, E_P - E), constant_values=N)
    attr_p = jnp.pad(edge_attr, ((0, E_P - E), (0, 0)))
    xg = xg_init
    for i in range(STEPS):
        y = _ymat(x, Wm[i][:D], bm[i][None, :])
        e_p = _emat(attr_p, Wm[i][D:])
        s = _p1(y, e_p, src_p)
        s_p = _repack(s)
        m = _p2(s_p, dst_p)
        m = m.reshape(ESH, CG, N, W)
        m2 = jnp.transpose(m, (0, 2, 1, 3)).reshape(ESH, N, D)
        xgn = (xg @ Wa[i][D:2 * D])[batch_ind]
        x = _update_x(x, m2, xgn, Wa[i][:D],
                      jnp.eye(D, dtype=jnp.float32), Wa[i][2 * D:], ba[i][None, :])
        gate = (x @ Wgate[i] + bgate[i])[:, 0]
        gmax = jax.ops.segment_max(gate, batch_ind, num_segments=G)
        gmax = jnp.where(jnp.isfinite(gmax), gmax, 0.0)
        eg = jnp.exp(gate - gmax[batch_ind])
        gsum = jax.ops.segment_sum(eg, batch_ind, num_segments=G)
        attn = eg / (gsum[batch_ind] + 1e-16)
        feat = _lrelu(x @ Wfeat[i] + bfeat[i])
        pooled = jax.ops.segment_sum(attn[:, None] * feat, batch_ind, num_segments=G)
        xg = _lrelu(pooled @ Wt[i][:D] + xg @ Wt[i][D:] + bt[i]) + xg
    return (x, xg)


# speedup vs baseline: 1.2193x; 1.2193x over previous
import functools

import jax
import jax.numpy as jnp
from jax import lax
from jax.experimental import pallas as pl
from jax.experimental.pallas import tpu as pltpu
from jax.experimental.pallas import tpu_sc as plsc

N = 10000
E = 320000
D = 128
DE = 16
G = 64
STEPS = 3
BLK = 1000
EBLK = 4000

NW = 32
E_P = 331776
ESZ = E_P // NW
EB1 = 384
NB1 = ESZ // EB1
CG = 16
ESH = 2
W = 8
EPS2 = E_P // ESH
EB2 = 2048
NB2 = EPS2 // EB2
SROWS = E_P // 16

_CP = pltpu.CompilerParams(needs_layout_passes=False)


def _lrelu(v):
    return jnp.where(v > 0, v, 0.01 * v)


def _p1_body(y_hbm, e_hbm, src_hbm, out_hbm, srcf, idxg, ybuf, sem):
    cid = lax.axis_index("c")
    sid = lax.axis_index("s")
    wid = sid * 2 + cid

    def blk(b, carry):
        e0 = wid * ESZ + b * EB1
        pltpu.sync_copy(src_hbm.at[pl.ds(e0, EB1)], srcf)
        for r in range(EB1 // 128):
            for i in range(8):
                idxg[r, pl.ds(i * 16, 16)] = srcf[pl.ds(r * 128 + i * 16, 16)]
        pltpu.sync_copy(e_hbm.at[pl.ds(e0, EB1)], ybuf)
        copies = [
            pltpu.async_copy(y_hbm.at[idxg.at[r]],
                             ybuf.at[pl.ds(r * 128, 128)], sem, add=True)
            for r in range(EB1 // 128)
        ]
        for cp in copies:
            cp.wait()
        pltpu.sync_copy(ybuf, out_hbm.at[pl.ds(e0, EB1)])
        return carry
    lax.fori_loop(0, NB1, blk, 0)


def _p1(y, e_p, src_p):
    kfn = functools.partial(
        pl.kernel,
        mesh=plsc.VectorSubcoreMesh(core_axis_name="c", subcore_axis_name="s"),
        compiler_params=_CP,
        out_type=jax.ShapeDtypeStruct((E_P, D), jnp.float32),
        scratch_types=[
            pltpu.VMEM((EB1,), jnp.int32),
            pltpu.VMEM((EB1 // 128, 128), jnp.int32),
            pltpu.VMEM((EB1, D), jnp.float32),
            pltpu.SemaphoreType.DMA,
        ],
    )(_p1_body)
    return kfn(y, e_p, src_p)


def _p2_body(s_hbm, dst_hbm, out_hbm, acc, dstf, sbuf):
    cid = lax.axis_index("c")
    sid = lax.axis_index("s")
    wid = sid * 2 + cid
    cg = wid % CG
    es = wid // CG
    i16 = lax.iota(jnp.int32, 16)
    neg = jnp.full((16,), -jnp.inf, jnp.float32)

    def init_body(i, carry):
        acc[pl.ds(i * 16, 16)] = neg
        return carry
    lax.fori_loop(0, N * W // 16, init_body, 0)

    def blk(b, carry):
        e0 = es * EPS2 + b * EB2
        pltpu.sync_copy(dst_hbm.at[pl.ds(e0, EB2)], dstf)
        row0 = pl.multiple_of(e0 // 16, 128)
        pltpu.sync_copy(s_hbm.at[cg, pl.ds(row0, EB2 // 16)], sbuf)

        def pair(j, eidx):
            i16_ = lax.iota(jnp.int32, 16)
            col_ = i16_ & 7
            swd_ = 1 - 2 * (i16_ // 8)
            eidx_sw = eidx + swd_
            fl = j * 16 + i16_
            rows = plsc.load_gather(dstf, [eidx])
            r_sw = plsc.load_gather(dstf, [eidx_sw])
            s = plsc.load_gather(sbuf, [fl >> 7, fl & 127])
            s_sw = plsc.load_gather(sbuf, [(fl >> 7), (fl & 127) + swd_ * 8])
            dup = rows == r_sw
            s = jnp.where(dup, jnp.maximum(s, s_sw), s)
            fidx = rows * 8 + col_
            cur = plsc.load_gather(acc, [fidx])
            s = jnp.maximum(s, cur)
            plsc.store_scatter(acc, [fidx], s)
            return eidx + 2
        lax.fori_loop(0, EB2 // 2, pair, i16 // 8)
        return carry
    lax.fori_loop(0, NB2, blk, 0)
    pltpu.sync_copy(acc, out_hbm.at[pl.ds(wid * N * W, N * W)])


def _p2(s_p, dst_p):
    kfn = functools.partial(
        pl.kernel,
        mesh=plsc.VectorSubcoreMesh(core_axis_name="c", subcore_axis_name="s"),
        compiler_params=_CP,
        out_type=jax.ShapeDtypeStruct((NW * N * W,), jnp.float32),
        scratch_types=[
            pltpu.VMEM((N * W,), jnp.float32),
            pltpu.VMEM((EB2,), jnp.int32),
            pltpu.VMEM((EB2 // 16, 128), jnp.float32),
        ],
    )(_p2_body)
    return kfn(s_p, dst_p)


def _ymat_body(x_ref, w_ref, b_ref, out_ref):
    out_ref[...] = (jnp.dot(x_ref[...], w_ref[...],
                            preferred_element_type=jnp.float32) + b_ref[...])


def _ymat(x, w, b):
    return pl.pallas_call(
        _ymat_body,
        grid=(N // BLK,),
        in_specs=[
            pl.BlockSpec((BLK, D), lambda i: (i, 0)),
            pl.BlockSpec((D, D), lambda i: (0, 0)),
            pl.BlockSpec((1, D), lambda i: (0, 0)),
        ],
        out_specs=pl.BlockSpec((BLK, D), lambda i: (i, 0)),
        out_shape=jax.ShapeDtypeStruct((N, D), jnp.float32),
    )(x, w, b)


def _emat_body(a_ref, w_ref, out_ref):
    out_ref[...] = jnp.dot(a_ref[...], w_ref[...],
                           preferred_element_type=jnp.float32)


def _emat(attr, w):
    return pl.pallas_call(
        _emat_body,
        grid=(E // EBLK,),
        in_specs=[
            pl.BlockSpec((EBLK, DE), lambda i: (i, 0)),
            pl.BlockSpec((DE, D), lambda i: (0, 0)),
        ],
        out_specs=pl.BlockSpec((EBLK, D), lambda i: (i, 0)),
        out_shape=jax.ShapeDtypeStruct((E, D), jnp.float32),
    )(attr, w)


def _update_x_body(x_ref, m_ref, xgn_ref, wa0_ref, wa1_ref, wa2_ref, ba_ref, out_ref):
    x = x_ref[...]
    m = jnp.max(m_ref[...], axis=0)
    agg = jnp.where(m > -jnp.inf, _lrelu(m), 0.0)
    acc = jnp.dot(x, wa0_ref[...], preferred_element_type=jnp.float32)
    acc += jnp.dot(xgn_ref[...], wa1_ref[...], preferred_element_type=jnp.float32)
    acc += jnp.dot(agg, wa2_ref[...], preferred_element_type=jnp.float32)
    acc += ba_ref[...]
    out_ref[...] = _lrelu(acc) + x


def _update_x(x, m2, xgn, wa0, wa1, wa2, ba):
    return pl.pallas_call(
        _update_x_body,
        grid=(N // BLK,),
        in_specs=[
            pl.BlockSpec((BLK, D), lambda i: (i, 0)),
            pl.BlockSpec((ESH, BLK, D), lambda i: (0, i, 0)),
            pl.BlockSpec((BLK, D), lambda i: (i, 0)),
            pl.BlockSpec((D, D), lambda i: (0, 0)),
            pl.BlockSpec((D, D), lambda i: (0, 0)),
            pl.BlockSpec((D, D), lambda i: (0, 0)),
            pl.BlockSpec((1, D), lambda i: (0, 0)),
        ],
        out_specs=pl.BlockSpec((BLK, D), lambda i: (i, 0)),
        out_shape=jax.ShapeDtypeStruct((N, D), jnp.float32),
    )(x, m2, xgn, wa0, wa1, wa2, ba)


def kernel(x, xg_init, edge_attr, Wm, bm, Wa, ba, Wgate, bgate, Wfeat, bfeat, Wt, bt,
           edge_index, batch_ind, num_graphs, data_lens):
    src_p = jnp.pad(edge_index[0], (0, E_P - E))
    dst_p = jnp.pad(edge_index[1], (0, E_P - E))
    xg = xg_init
    for i in range(STEPS):
        y = _ymat(x, Wm[i][:D], bm[i][None, :])
        e = _emat(edge_attr, Wm[i][D:])
        e_p = jnp.concatenate(
            [e, jnp.full((E_P - E, D), -jnp.inf, jnp.float32)], axis=0)
        s = _p1(y, e_p, src_p)
        s_p = jnp.transpose(
            s.reshape(E_P // 16, 16, CG, W), (2, 0, 1, 3)).reshape(CG, SROWS, 128)
        m = _p2(s_p, dst_p)
        m = m.reshape(ESH, CG, N, W)
        m2 = jnp.transpose(m, (0, 2, 1, 3)).reshape(ESH, N, D)
        xgn = (xg @ Wa[i][D:2 * D])[batch_ind]
        x = _update_x(x, m2, xgn, Wa[i][:D],
                      jnp.eye(D, dtype=jnp.float32), Wa[i][2 * D:], ba[i][None, :])
        gate = (x @ Wgate[i] + bgate[i])[:, 0]
        gmax = jax.ops.segment_max(gate, batch_ind, num_segments=G)
        gmax = jnp.where(jnp.isfinite(gmax), gmax, 0.0)
        eg = jnp.exp(gate - gmax[batch_ind])
        gsum = jax.ops.segment_sum(eg, batch_ind, num_segments=G)
        attn = eg / (gsum[batch_ind] + 1e-16)
        feat = _lrelu(x @ Wfeat[i] + bfeat[i])
        pooled = jax.ops.segment_sum(attn[:, None] * feat, batch_ind, num_segments=G)
        xg = _lrelu(pooled @ Wt[i][:D] + xg @ Wt[i][D:] + bt[i]) + xg
    return (x, xg)
